# baseline (speedup 1.0000x reference)
import jax
import jax.numpy as jnp
from jax import lax
from jax.experimental import pallas as pl
from jax.experimental.pallas import tpu as pltpu
from jax.experimental.pallas import tpu_sc as plsc

N = 10000
E = 160000
D_IN = 256
D_HID = 512

_NC = 2
_NS = 16
_CH = 64
_R = 2
_NCH0 = 160
_TCH = _NS * _NCH0
_TCH_PAD = _TCH
_E_PAD = _TCH * _CH
_N_PAD = 10112
_RPT = _N_PAD // _NS
_M_BLK = 2528


def _make_agg_body(nchk):
    half = _R // 2

    def body(*refs):
        tables = refs[:nchk]
        src_hbm, dst_hbm, out_hbm = refs[nchk:nchk + 3]
        scr = refs[nchk + 3:]
        src_v = scr[0]
        dst_v = scr[1:1 + _R]
        rows = scr[1 + _R:1 + 2 * _R]
        zbuf = scr[1 + 2 * _R]
        aggm = scr[2 + 2 * _R]
        semg = scr[3 + 2 * _R:3 + 3 * _R]
        semi = scr[3 + 3 * _R:3 + 4 * _R]
        c = lax.axis_index("c")
        s = lax.axis_index("s")
        my = pl.ds(s * _RPT, _RPT)

        def gath(tab, k, b):
            pltpu.async_copy(tab.at[src_v.at[k]], rows[b], semg[b])

        def wait_g(tab, k, b):
            pltpu.make_async_copy(tab.at[src_v.at[k]], rows[b], semg[b]).wait()

        def load_d(k, b):
            pltpu.async_copy(dst_hbm.at[s * _NCH0 + k], dst_v[b], semi[b])

        def wait_d(k, b):
            pltpu.make_async_copy(dst_hbm.at[s * _NCH0 + k], dst_v[b],
                                  semi[b]).wait()

        def scat(b):
            pltpu.sync_copy(rows[b], aggm.at[dst_v[b]], add=True)

        def zero_my():
            for piece in range(_RPT // _CH):
                pltpu.sync_copy(
                    zbuf, aggm.at[pl.ds(s * _RPT + piece * _CH, _CH)])
            rem = _RPT % _CH
            if rem:
                pltpu.sync_copy(
                    zbuf.at[pl.ds(0, rem)],
                    aggm.at[pl.ds(s * _RPT + _RPT - rem, rem)])

        @pl.when(c == 0)
        def _work():
            def zfill(i, carry):
                for j in range(8):
                    zbuf[i, pl.ds(j * 16, 16)] = jnp.zeros((16,), jnp.float32)
                return carry

            lax.fori_loop(0, _CH, zfill, 0)
            pltpu.sync_copy(src_hbm.at[pl.ds(s * _NCH0, _NCH0)], src_v)
            zero_my()
            plsc.subcore_barrier()
            for cc in range(nchk):
                tab = tables[cc]
                for b in range(_R):
                    gath(tab, b, b)
                    load_d(b, b)

                def step(it, carry):
                    base = _R * it
                    for j in range(_R):
                        k = base + j
                        wait_g(tab, k, j)
                        wait_d(k, j)
                        scat(j)
                        gath(tab, k + _R, j)
                        load_d(k + _R, j)
                    return carry

                lax.fori_loop(0, _NCH0 // _R - 1, step, 0)
                for j in range(_R):
                    k = _NCH0 - _R + j
                    wait_g(tab, k, j)
                    wait_d(k, j)
                    scat(j)

                plsc.subcore_barrier()
                pltpu.sync_copy(aggm.at[my], out_hbm.at[cc, my])
                if cc + 1 < nchk:
                    zero_my()
                    plsc.subcore_barrier()
    return body


def _agg_sc(tables, src2, dst2):
    nchk = len(tables)
    mesh = plsc.VectorSubcoreMesh(core_axis_name="c", subcore_axis_name="s")
    f = pl.kernel(
        _make_agg_body(nchk),
        mesh=mesh,
        out_type=jax.ShapeDtypeStruct((nchk, _N_PAD, 128), jnp.float32),
        scratch_types=(
            [pltpu.VMEM((_NCH0, _CH), jnp.int32)]
            + [pltpu.VMEM((_CH,), jnp.int32) for _ in range(_R)]
            + [pltpu.VMEM((_CH, 128), jnp.float32) for _ in range(_R)]
            + [pltpu.VMEM((_CH, 128), jnp.float32)]
            + [pltpu.VMEM_SHARED((_N_PAD, 128), jnp.float32)]
            + [pltpu.SemaphoreType.DMA for _ in range(2 * _R)]
        ),
    )
    return f(*tables, src2, dst2)


def _mlp1_body(x2_ref, p_ref, wa_ref, ba_ref, wb_ref, bb_ref, o_ref):
    xin = jnp.concatenate([x2_ref[0], x2_ref[1]], axis=1)
    agg = jnp.concatenate([p_ref[0], p_ref[1]], axis=1)
    h = xin + agg
    h = jnp.dot(h, wa_ref[...], preferred_element_type=jnp.float32) + ba_ref[...]
    h = jnp.maximum(h, 0.0)
    h = jnp.dot(h, wb_ref[...], preferred_element_type=jnp.float32) + bb_ref[...]
    h = jnp.maximum(h, 0.0)
    for c in range(4):
        o_ref[c] = h[:, c * 128:(c + 1) * 128]


def _mlp1(x2, p, Wa, ba, Wb, bb):
    grid = (_N_PAD // _M_BLK,)
    return pl.pallas_call(
        _mlp1_body,
        grid=grid,
        in_specs=[
            pl.BlockSpec((2, _M_BLK, 128), lambda i: (0, i, 0)),
            pl.BlockSpec((2, _M_BLK, 128), lambda i: (0, i, 0)),
            pl.BlockSpec((D_IN, D_HID), lambda i: (0, 0)),
            pl.BlockSpec((1, D_HID), lambda i: (0, 0)),
            pl.BlockSpec((D_HID, D_HID), lambda i: (0, 0)),
            pl.BlockSpec((1, D_HID), lambda i: (0, 0)),
        ],
        out_specs=pl.BlockSpec((4, _M_BLK, 128), lambda i: (0, i, 0)),
        out_shape=jax.ShapeDtypeStruct((4, _N_PAD, 128), jnp.float32),
    )(x2, p, Wa, ba.reshape(1, -1), Wb, bb.reshape(1, -1))


def _mlp2_body(h2_ref, q_ref, wa_ref, ba_ref, wb_ref, bb_ref, o_ref):
    xin = jnp.concatenate([h2_ref[c] for c in range(4)], axis=1)
    agg = jnp.concatenate([q_ref[c] for c in range(4)], axis=1)
    h = xin + agg
    h = jnp.dot(h, wa_ref[...], preferred_element_type=jnp.float32) + ba_ref[...]
    h = jnp.maximum(h, 0.0)
    h = jnp.dot(h, wb_ref[...], preferred_element_type=jnp.float32) + bb_ref[...]
    o_ref[...] = jnp.maximum(h, 0.0)


def _mlp2(h2, q, Wa, ba, Wb, bb):
    grid = (_N_PAD // _M_BLK,)
    return pl.pallas_call(
        _mlp2_body,
        grid=grid,
        in_specs=[
            pl.BlockSpec((4, _M_BLK, 128), lambda i: (0, i, 0)),
            pl.BlockSpec((4, _M_BLK, 128), lambda i: (0, i, 0)),
            pl.BlockSpec((D_HID, D_HID), lambda i: (0, 0)),
            pl.BlockSpec((1, D_HID), lambda i: (0, 0)),
            pl.BlockSpec((D_HID, D_HID), lambda i: (0, 0)),
            pl.BlockSpec((1, D_HID), lambda i: (0, 0)),
        ],
        out_specs=pl.BlockSpec((_M_BLK, D_HID), lambda i: (i, 0)),
        out_shape=jax.ShapeDtypeStruct((_N_PAD, D_HID), jnp.float32),
    )(h2, q, Wa, ba.reshape(1, -1), Wb, bb.reshape(1, -1))


def kernel(x, edge_index, W1a, b1a, W1b, b1b, W2a, b2a, W2b, b2b):
    idx = edge_index.astype(jnp.int32)
    pad = _E_PAD - E
    tail = (_TCH_PAD - _TCH) * _CH
    src2 = jnp.concatenate(
        [idx[0], jnp.zeros((pad + tail,), jnp.int32)]).reshape(_TCH_PAD, _CH)
    pad_dst = N + (jnp.arange(pad + tail, dtype=jnp.int32) % (_N_PAD - N))
    dst2 = jnp.concatenate([idx[1], pad_dst]).reshape(_TCH_PAD, _CH)

    xp = jnp.pad(x, ((0, _N_PAD - N), (0, 0)))
    x2 = xp.reshape(_N_PAD, 2, 128).transpose(1, 0, 2)

    p = _agg_sc([x2[0], x2[1]], src2, dst2)
    h2 = _mlp1(x2, p, W1a, b1a, W1b, b1b)
    q = _agg_sc([h2[0], h2[1], h2[2], h2[3]], src2, dst2)
    out = _mlp2(h2, q, W2a, b2a, W2b, b2b)
    return out[:N]


# speedup vs baseline: 1.2330x; 1.2330x over previous
import jax
import jax.numpy as jnp
from jax import lax
from jax.experimental import pallas as pl
from jax.experimental.pallas import tpu as pltpu
from jax.experimental.pallas import tpu_sc as plsc

N = 10000
E = 160000
D_IN = 256
D_HID = 512

_NC = 2
_NS = 16
_CH = 64
_R = 2
_NCH0 = 120
_NCH1 = 40
_TCH = _NS * (_NCH0 + _NCH1)
_TCH_PAD = _TCH + (_NCH0 - _NCH1)
_E_PAD = _TCH * _CH
_N_PAD = 10112
_RPT = _N_PAD // _NS
_M_BLK = 2528


def _make_agg_body(nchk):
    half = _R // 2

    def body(*refs):
        tables = refs[:nchk]
        src_hbm, dst_hbm, z_hbm, out_hbm = refs[nchk:nchk + 4]
        scr = refs[nchk + 4:]
        src_v, dst_v = scr[0], scr[1]
        rows = scr[2:2 + _R]
        aggm = scr[2 + _R]
        semg = scr[3 + _R:3 + 2 * _R]
        c = lax.axis_index("c")
        s = lax.axis_index("s")
        my = pl.ds(s * _RPT, _RPT)
        n = jnp.where(c == 0, _NCH0, _NCH1)
        base_chunk = jnp.where(c == 0, s * _NCH0, _NS * _NCH0 + s * _NCH1)

        def gath(tab, k, b):
            pltpu.async_copy(tab.at[src_v.at[k]], rows[b], semg[b])

        def wait_g(tab, k, b):
            pltpu.make_async_copy(tab.at[src_v.at[k]], rows[b], semg[b]).wait()

        def scat(k, b):
            pltpu.sync_copy(rows[b], aggm.at[dst_v.at[k]], add=True)

        pltpu.sync_copy(src_hbm.at[pl.ds(base_chunk, _NCH0)], src_v)
        pltpu.sync_copy(dst_hbm.at[pl.ds(base_chunk, _NCH0)], dst_v)
        pltpu.sync_copy(z_hbm, aggm.at[my])
        plsc.subcore_barrier()
        for cc in range(nchk):
            tab = tables[cc]
            for b in range(_R):
                gath(tab, b, b)

            def step(it, carry):
                base = _R * it
                for j in range(_R):
                    k = base + j
                    wait_g(tab, k, j)
                    scat(k, j)
                    gath(tab, k + _R, j)
                return carry

            lax.fori_loop(0, n // _R - 1, step, 0)
            for j in range(_R):
                k = n - _R + j
                wait_g(tab, k, j)
                scat(k, j)

            plsc.subcore_barrier()
            pltpu.sync_copy(aggm.at[my], out_hbm.at[cc, c, my])
            if cc + 1 < nchk:
                pltpu.sync_copy(z_hbm, aggm.at[my])
                plsc.subcore_barrier()
    return body


def _agg_sc(tables, src2, dst2, z):
    nchk = len(tables)
    mesh = plsc.VectorSubcoreMesh(core_axis_name="c", subcore_axis_name="s")
    f = pl.kernel(
        _make_agg_body(nchk),
        mesh=mesh,
        out_type=jax.ShapeDtypeStruct((nchk, _NC, _N_PAD, 128), jnp.float32),
        scratch_types=(
            [pltpu.VMEM((_NCH0, _CH), jnp.int32)] * 2
            + [pltpu.VMEM((_CH, 128), jnp.float32) for _ in range(_R)]
            + [pltpu.VMEM_SHARED((_N_PAD, 128), jnp.float32)]
            + [pltpu.SemaphoreType.DMA for _ in range(_R)]
        ),
    )
    return f(*tables, src2, dst2, z)


def _mlp1_body(x2_ref, p_ref, wa_ref, ba_ref, wb_ref, bb_ref, o_ref):
    xin = jnp.concatenate([x2_ref[0], x2_ref[1]], axis=1)
    agg = jnp.concatenate([p_ref[0, 0] + p_ref[0, 1],
                           p_ref[1, 0] + p_ref[1, 1]], axis=1)
    h = xin + agg
    h = jnp.dot(h, wa_ref[...], preferred_element_type=jnp.float32) + ba_ref[...]
    h = jnp.maximum(h, 0.0)
    h = jnp.dot(h, wb_ref[...], preferred_element_type=jnp.float32) + bb_ref[...]
    h = jnp.maximum(h, 0.0)
    for c in range(4):
        o_ref[c] = h[:, c * 128:(c + 1) * 128]


def _mlp1(x2, p, Wa, ba, Wb, bb):
    grid = (_N_PAD // _M_BLK,)
    return pl.pallas_call(
        _mlp1_body,
        grid=grid,
        in_specs=[
            pl.BlockSpec((2, _M_BLK, 128), lambda i: (0, i, 0)),
            pl.BlockSpec((2, 2, _M_BLK, 128), lambda i: (0, 0, i, 0)),
            pl.BlockSpec((D_IN, D_HID), lambda i: (0, 0)),
            pl.BlockSpec((1, D_HID), lambda i: (0, 0)),
            pl.BlockSpec((D_HID, D_HID), lambda i: (0, 0)),
            pl.BlockSpec((1, D_HID), lambda i: (0, 0)),
        ],
        out_specs=pl.BlockSpec((4, _M_BLK, 128), lambda i: (0, i, 0)),
        out_shape=jax.ShapeDtypeStruct((4, _N_PAD, 128), jnp.float32),
    )(x2, p, Wa, ba.reshape(1, -1), Wb, bb.reshape(1, -1))


def _mlp2_body(h2_ref, q_ref, wa_ref, ba_ref, wb_ref, bb_ref, o_ref):
    xin = jnp.concatenate([h2_ref[c] for c in range(4)], axis=1)
    agg = jnp.concatenate([q_ref[c, 0] + q_ref[c, 1] for c in range(4)], axis=1)
    h = xin + agg
    h = jnp.dot(h, wa_ref[...], preferred_element_type=jnp.float32) + ba_ref[...]
    h = jnp.maximum(h, 0.0)
    h = jnp.dot(h, wb_ref[...], preferred_element_type=jnp.float32) + bb_ref[...]
    o_ref[...] = jnp.maximum(h, 0.0)


def _mlp2(h2, q, Wa, ba, Wb, bb):
    grid = (_N_PAD // _M_BLK,)
    return pl.pallas_call(
        _mlp2_body,
        grid=grid,
        in_specs=[
            pl.BlockSpec((4, _M_BLK, 128), lambda i: (0, i, 0)),
            pl.BlockSpec((4, 2, _M_BLK, 128), lambda i: (0, 0, i, 0)),
            pl.BlockSpec((D_HID, D_HID), lambda i: (0, 0)),
            pl.BlockSpec((1, D_HID), lambda i: (0, 0)),
            pl.BlockSpec((D_HID, D_HID), lambda i: (0, 0)),
            pl.BlockSpec((1, D_HID), lambda i: (0, 0)),
        ],
        out_specs=pl.BlockSpec((_M_BLK, D_HID), lambda i: (i, 0)),
        out_shape=jax.ShapeDtypeStruct((_N_PAD, D_HID), jnp.float32),
    )(h2, q, Wa, ba.reshape(1, -1), Wb, bb.reshape(1, -1))


def kernel(x, edge_index, W1a, b1a, W1b, b1b, W2a, b2a, W2b, b2b):
    idx = edge_index.astype(jnp.int32)
    pad = _E_PAD - E
    tail = (_TCH_PAD - _TCH) * _CH
    src2 = jnp.concatenate(
        [idx[0], jnp.zeros((pad + tail,), jnp.int32)]).reshape(_TCH_PAD, _CH)
    pad_dst = N + (jnp.arange(pad + tail, dtype=jnp.int32) % (_N_PAD - N))
    dst2 = jnp.concatenate([idx[1], pad_dst]).reshape(_TCH_PAD, _CH)
    z = jnp.zeros((_RPT, 128), jnp.float32)

    xp = jnp.pad(x, ((0, _N_PAD - N), (0, 0)))
    x2 = xp.reshape(_N_PAD, 2, 128).transpose(1, 0, 2)

    p = _agg_sc([x2[0], x2[1]], src2, dst2, z)
    h2 = _mlp1(x2, p, W1a, b1a, W1b, b1b)
    q = _agg_sc([h2[0], h2[1], h2[2], h2[3]], src2, dst2, z)
    out = _mlp2(h2, q, W2a, b2a, W2b, b2b)
    return out[:N]
